# baseline (speedup 1.0000x reference)
import functools

import jax
import jax.numpy as jnp
from jax import lax
from jax.experimental import pallas as pl
from jax.experimental.pallas import tpu as pltpu
from jax.experimental.pallas import tpu_sc as plsc

D_MODEL = 768
D_VOCAB = 100000
B, P = 4, 2048
N_TOK = B * P

_NC, _NS = 2, 16
_NW = _NC * _NS
_TPW = N_TOK // _NW
_WPB = P // _TPW
_CH = 128


_NBUF = 4
_CHS = 32
_NCHS = _TPW // _CHS


def _gather_body(table_hbm, idx_hbm, out_hbm, idx_v, b0, b1, b2, b3,
                 g0, g1, g2, g3, w0, w1, w2, w3):
    wid = lax.axis_index("s") * _NC + lax.axis_index("c")
    b = wid // _WPB
    p0 = (wid % _WPB) * _TPW
    bufs = (b0, b1, b2, b3)
    gsems = (g0, g1, g2, g3)
    wsems = (w0, w1, w2, w3)
    pltpu.sync_copy(idx_hbm.at[b, pl.ds(p0, _TPW)], idx_v)

    def gather(j):
        k = j % _NBUF
        return pltpu.async_copy(
            table_hbm.at[idx_v.at[pl.ds(j * _CHS, _CHS)]], bufs[k], gsems[k]
        )

    def write(j):
        k = j % _NBUF
        return pltpu.async_copy(
            bufs[k], out_hbm.at[b, pl.ds(p0 + j * _CHS, _CHS)], wsems[k]
        )

    gops = [None] * _NCHS
    wops = [None] * _NCHS
    gops[0] = gather(0)
    gops[1] = gather(1)
    for j in range(_NCHS):
        gops[j].wait()
        wops[j] = write(j)
        nxt = j + 2
        if nxt < _NCHS:
            if nxt - _NBUF >= 0:
                wops[nxt - _NBUF].wait()
            gops[nxt] = gather(nxt)
    for j in range(_NCHS - _NBUF, _NCHS):
        wops[j].wait()


def _gather(W_T, x):
    mesh = plsc.VectorSubcoreMesh(core_axis_name="c", subcore_axis_name="s")
    f = functools.partial(
        pl.kernel,
        mesh=mesh,
        out_type=jax.ShapeDtypeStruct((B, P, D_MODEL), jnp.float32),
        scratch_types=[
            pltpu.VMEM((_TPW,), jnp.int32),
            pltpu.VMEM((_CHS, D_MODEL), jnp.float32),
            pltpu.VMEM((_CHS, D_MODEL), jnp.float32),
            pltpu.VMEM((_CHS, D_MODEL), jnp.float32),
            pltpu.VMEM((_CHS, D_MODEL), jnp.float32),
            pltpu.SemaphoreType.DMA,
            pltpu.SemaphoreType.DMA,
            pltpu.SemaphoreType.DMA,
            pltpu.SemaphoreType.DMA,
            pltpu.SemaphoreType.DMA,
            pltpu.SemaphoreType.DMA,
            pltpu.SemaphoreType.DMA,
            pltpu.SemaphoreType.DMA,
        ],
    )(_gather_body)
    return f(W_T, x)


def kernel(x, W_E):
    W_T = W_E.T
    return _gather(W_T, x.astype(jnp.int32))


# speedup vs baseline: 1.0108x; 1.0108x over previous
import functools

import jax
import jax.numpy as jnp
from jax import lax
from jax.experimental import pallas as pl
from jax.experimental.pallas import tpu as pltpu
from jax.experimental.pallas import tpu_sc as plsc

D_MODEL = 768
D_VOCAB = 100000
B, P = 4, 2048
N_TOK = B * P

_NC, _NS = 2, 16
_NW = _NC * _NS
_TPW = N_TOK // _NW
_WPB = P // _TPW
_CH = 128


def _gather_body(table_hbm, idx_hbm, out_hbm, idx_v, rows_v, sem):
    wid = lax.axis_index("s") * _NC + lax.axis_index("c")
    b = wid // _WPB
    p0 = (wid % _WPB) * _TPW
    pltpu.sync_copy(idx_hbm.at[b, pl.ds(p0, _TPW)], idx_v)
    for j in range(_TPW // _CH):
        p = p0 + j * _CH
        pltpu.async_copy(
            table_hbm.at[idx_v.at[pl.ds(j * _CH, _CH)]], rows_v, sem
        ).wait()
        pltpu.sync_copy(rows_v, out_hbm.at[b, pl.ds(p, _CH)])


def _gather(W_T, x):
    mesh = plsc.VectorSubcoreMesh(core_axis_name="c", subcore_axis_name="s")
    f = functools.partial(
        pl.kernel,
        mesh=mesh,
        out_type=jax.ShapeDtypeStruct((B, P, D_MODEL), jnp.float32),
        scratch_types=[
            pltpu.VMEM((_TPW,), jnp.int32),
            pltpu.VMEM((_CH, D_MODEL), jnp.float32),
            pltpu.SemaphoreType.DMA,
        ],
    )(_gather_body)
    return f(W_T, x)


def kernel(x, W_E):
    W_T = W_E.T
    return _gather(W_T, x.astype(jnp.int32))
